# Initial kernel scaffold; baseline (speedup 1.0000x reference)
import jax
import jax.numpy as jnp
from jax.experimental import pallas as pl


def kernel(x, edge_index, edge_type, W, G_sender, G_receiver):
    raise NotImplementedError("write your pallas kernel here")


# speedup vs baseline: 6.4390x; 6.4390x over previous
import jax
import jax.numpy as jnp
from jax import lax
from jax.experimental import pallas as pl
from jax.experimental.pallas import tpu as pltpu
from jax.experimental.pallas import tpu_sc as plsc

N = 10000
E = 320000
D = 128
R = 200

NC = 2
NS = 16
NW = NC * NS

HALF_N = N // NC
TBL = 1 << 20
TRASH = HALF_N * R

ROWS = E // 100
A_CHUNK = 40
A_STEPS = ROWS // (NS * A_CHUNK)
C_CHUNK = 8
C_STEPS = ROWS // (NW * C_CHUNK)
C_EXTRA = ROWS - NW * C_CHUNK * C_STEPS

ZROWS = 624


def _prep_kernel(src_ref, dst_ref, et_ref, hist_ref, gidx_ref):
    src = src_ref[...]
    dst = dst_ref[...]
    t = et_ref[...]
    gidx_ref[...] = src * R + t
    hist_ref[0] = jnp.where(dst < HALF_N, dst * R + t, TRASH)
    hist_ref[1] = jnp.where(dst >= HALF_N, (dst - HALF_N) * R + t, TRASH)


def _hist_sc_kernel(hist_idx, count_out, table, idxbuf, zbuf, ones):
    c = lax.axis_index("c")
    s = lax.axis_index("s")

    def zb(i, _):
        zbuf[pl.ds(pl.multiple_of(i * 16, 16), 16)] = jnp.zeros(
            (16,), jnp.float32)
        return _
    lax.fori_loop(0, 512, zb, None)
    for i in range(7):
        ones[pl.ds(i * 16, 16)] = jnp.ones((16,), jnp.float32)

    zwords = TBL // NS
    for k in range(zwords // 8192):
        pltpu.sync_copy(
            zbuf,
            table.at[pl.ds(pl.multiple_of(s * zwords + k * 8192, 8192), 8192)])
    plsc.subcore_barrier()

    def chunk(k, _):
        rowbase = pl.multiple_of((k * NS + s) * A_CHUNK, 8)
        pltpu.sync_copy(hist_idx.at[c, pl.ds(rowbase, A_CHUNK)], idxbuf)
        for j in range(A_CHUNK):
            pltpu.sync_copy(ones.at[pl.ds(0, 100)], table.at[idxbuf.at[j]],
                            add=True)
        return _
    lax.fori_loop(0, A_STEPS, chunk, None)
    plsc.subcore_barrier()

    base = pl.multiple_of(c * TBL + s * zwords, 8192)
    pltpu.sync_copy(table.at[pl.ds(pl.multiple_of(s * zwords, 8192), zwords)],
                    count_out.at[pl.ds(base, zwords)])


def _dense_kernel(count_ref, x_ref, w_ref, gs_ref, gr_ref, h_ref, gate_ref):
    b = jnp.dot(gr_ref[...], gs_ref[...].T,
                preferred_element_type=jnp.float32)
    gate_ref[...] = jax.nn.sigmoid(
        jnp.dot(count_ref[...], b, preferred_element_type=jnp.float32))
    h_ref[...] = jnp.dot(x_ref[...], w_ref[...],
                         preferred_element_type=jnp.float32)


def _agg_sc_kernel(h_hbm, gflat, gate_idx, src2d, dst2d, part_out,
                   acc, rows, gbuf, gidxbuf, sbuf, dbuf, zbuf, sem):
    c = lax.axis_index("c")
    s = lax.axis_index("s")
    wid = s * NC + c

    def zb(r, _):
        for f in range(D // 16):
            zbuf[r, pl.ds(f * 16, 16)] = jnp.zeros((16,), jnp.float32)
        return _
    lax.fori_loop(0, 48, zb, None)
    zbase = pl.multiple_of(s * ZROWS, 8)
    for k in range(ZROWS // 48):
        pltpu.sync_copy(zbuf, acc.at[pl.ds(zbase + k * 48, 48)])
    @pl.when(s == NS - 1)
    def _():
        pltpu.sync_copy(zbuf.at[pl.ds(0, 16)], acc.at[pl.ds(N - 16, 16)])
    plsc.subcore_barrier()

    def do_chunk(rowbase):
        pltpu.sync_copy(gate_idx.at[pl.ds(rowbase, C_CHUNK)], gidxbuf)
        pltpu.sync_copy(src2d.at[pl.ds(rowbase, C_CHUNK)], sbuf)
        pltpu.sync_copy(dst2d.at[pl.ds(rowbase, C_CHUNK)], dbuf)
        for j in range(C_CHUNK):
            b = j % 2
            pltpu.async_copy(h_hbm.at[sbuf.at[j]], rows.at[b], sem).wait()
            pltpu.async_copy(gflat.at[gidxbuf.at[j]],
                             gbuf.at[j, pl.ds(0, 100)], sem).wait()

            def scale(r, _):
                lane = r & 15
                gv = gbuf[j, pl.ds(pl.multiple_of(r - lane, 16), 16)]
                g = gv.at[jnp.full((16,), lane, jnp.int32)].get(
                    mode="promise_in_bounds")
                for f in range(D // 16):
                    rows[b, r, pl.ds(f * 16, 16)] = (
                        rows[b, r, pl.ds(f * 16, 16)] * g)
                return _
            lax.fori_loop(0, 100, scale, None)

            pltpu.sync_copy(rows.at[b], acc.at[dbuf.at[j]], add=True)

    def chunk(k, _):
        do_chunk(pl.multiple_of((k * NW + wid) * C_CHUNK, 8))
        return _
    lax.fori_loop(0, C_STEPS, chunk, None)
    @pl.when(wid < C_EXTRA // C_CHUNK)
    def _():
        do_chunk(pl.multiple_of((C_STEPS * NW + wid) * C_CHUNK, 8))
    plsc.subcore_barrier()

    zbase2 = pl.multiple_of(s * ZROWS, 8)
    pltpu.sync_copy(acc.at[pl.ds(zbase2, ZROWS)],
                    part_out.at[c, pl.ds(zbase2, ZROWS)])
    @pl.when(s == NS - 1)
    def _():
        pltpu.sync_copy(acc.at[pl.ds(N - 16, 16)],
                        part_out.at[c, pl.ds(N - 16, 16)])


def _final_kernel(part_ref, out_ref):
    out_ref[...] = jnp.maximum(part_ref[0] + part_ref[1], 0.0)


@jax.jit
def kernel(x, edge_index, edge_type, W, G_sender, G_receiver):
    src2d = edge_index[0].reshape(ROWS, 100)
    dst2d = edge_index[1].reshape(ROWS, 100)
    et2d = edge_type.reshape(ROWS, 100)

    hist_idx, gate_idx = pl.pallas_call(
        _prep_kernel,
        out_shape=[
            jax.ShapeDtypeStruct((NC, ROWS, 100), jnp.int32),
            jax.ShapeDtypeStruct((ROWS, 100), jnp.int32),
        ],
    )(src2d, dst2d, et2d)

    mesh = plsc.VectorSubcoreMesh(core_axis_name="c", subcore_axis_name="s")

    count_raw = pl.kernel(
        _hist_sc_kernel,
        out_type=jax.ShapeDtypeStruct((NC * TBL,), jnp.float32),
        mesh=mesh,
        scratch_types=[
            pltpu.VMEM_SHARED((TBL,), jnp.float32),
            pltpu.VMEM((A_CHUNK, 100), jnp.int32),
            pltpu.VMEM((8192,), jnp.float32),
            pltpu.VMEM((112,), jnp.float32),
        ],
    )(hist_idx)

    count = jnp.concatenate(
        [count_raw[:HALF_N * R], count_raw[TBL:TBL + HALF_N * R]]
    ).reshape(N, R)

    h, gate_table = pl.pallas_call(
        _dense_kernel,
        grid=(10,),
        in_specs=[
            pl.BlockSpec((N // 10, R), lambda m: (m, 0)),
            pl.BlockSpec((N // 10, D), lambda m: (m, 0)),
            pl.BlockSpec((D, D), lambda m: (0, 0)),
            pl.BlockSpec((R, 100), lambda m: (0, 0)),
            pl.BlockSpec((R, 100), lambda m: (0, 0)),
        ],
        out_specs=[
            pl.BlockSpec((N // 10, D), lambda m: (m, 0)),
            pl.BlockSpec((N // 10, R), lambda m: (m, 0)),
        ],
        out_shape=[
            jax.ShapeDtypeStruct((N, D), jnp.float32),
            jax.ShapeDtypeStruct((N, R), jnp.float32),
        ],
    )(count, x, W, G_sender, G_receiver)

    gflat = gate_table.reshape(N * R)

    partials = pl.kernel(
        _agg_sc_kernel,
        out_type=jax.ShapeDtypeStruct((NC, N, D), jnp.float32),
        mesh=mesh,
        scratch_types=[
            pltpu.VMEM_SHARED((N, D), jnp.float32),
            pltpu.VMEM((2, 100, D), jnp.float32),
            pltpu.VMEM((C_CHUNK, 128), jnp.float32),
            pltpu.VMEM((C_CHUNK, 100), jnp.int32),
            pltpu.VMEM((C_CHUNK, 100), jnp.int32),
            pltpu.VMEM((C_CHUNK, 100), jnp.int32),
            pltpu.VMEM((48, D), jnp.float32),
            pltpu.SemaphoreType.DMA,
        ],
    )(h, gflat, gate_idx, src2d, dst2d)

    out = pl.pallas_call(
        _final_kernel,
        grid=(10,),
        in_specs=[pl.BlockSpec((NC, N // 10, D), lambda m: (0, m, 0))],
        out_specs=pl.BlockSpec((N // 10, D), lambda m: (m, 0)),
        out_shape=jax.ShapeDtypeStruct((N, D), jnp.float32),
    )(partials)
    return out


# speedup vs baseline: 8.6175x; 1.3383x over previous
import jax
import jax.numpy as jnp
from jax import lax
from jax.experimental import pallas as pl
from jax.experimental.pallas import tpu as pltpu
from jax.experimental.pallas import tpu_sc as plsc

N = 10000
E = 320000
D = 128
R = 200

NC = 2
NS = 16
NW = NC * NS

HALF_N = N // NC
TBL = 1 << 20
TRASH = HALF_N * R

ROWS = E // 100
A_CHUNK = 40
A_STEPS = ROWS // (NS * A_CHUNK)
C_CHUNK = 8
C_STEPS = ROWS // (NW * C_CHUNK)
C_EXTRA = ROWS - NW * C_CHUNK * C_STEPS

ZROWS = 624


def _prep_kernel(src_ref, dst_ref, et_ref, hist_ref, gidx_ref):
    src = src_ref[...]
    dst = dst_ref[...]
    t = et_ref[...]
    gidx_ref[...] = src * R + t
    hist_ref[0] = jnp.where(dst < HALF_N, dst * R + t, TRASH)
    hist_ref[1] = jnp.where(dst >= HALF_N, (dst - HALF_N) * R + t, TRASH)


def _hist_sc_kernel(hist_idx, count_out, table, idxbuf, zbuf, ones, sem):
    c = lax.axis_index("c")
    s = lax.axis_index("s")

    def zb(i, _):
        zbuf[pl.ds(pl.multiple_of(i * 16, 16), 16)] = jnp.zeros(
            (16,), jnp.float32)
        return _
    lax.fori_loop(0, 512, zb, None)
    for i in range(7):
        ones[pl.ds(i * 16, 16)] = jnp.ones((16,), jnp.float32)

    zwords = TBL // NS
    for k in range(zwords // 8192):
        pltpu.sync_copy(
            zbuf,
            table.at[pl.ds(pl.multiple_of(s * zwords + k * 8192, 8192), 8192)])
    plsc.subcore_barrier()

    rowbase = pl.multiple_of(s * (ROWS // NS), 8)
    pltpu.sync_copy(hist_idx.at[c, pl.ds(rowbase, ROWS // NS)], idxbuf)
    for w in range(ROWS // NS // A_CHUNK):
        ds = [pltpu.async_copy(ones.at[pl.ds(0, 100)],
                               table.at[idxbuf.at[w * A_CHUNK + j]],
                               sem, add=True)
              for j in range(A_CHUNK)]
        for d in ds:
            d.wait()
    plsc.subcore_barrier()

    base = pl.multiple_of(c * TBL + s * zwords, 8192)
    pltpu.sync_copy(table.at[pl.ds(pl.multiple_of(s * zwords, 8192), zwords)],
                    count_out.at[pl.ds(base, zwords)])


def _dense_kernel(count_ref, x_ref, w_ref, gs_ref, gr_ref, h_ref, gate_ref):
    b = jnp.dot(gr_ref[...], gs_ref[...].T,
                preferred_element_type=jnp.float32)
    gate_ref[...] = jax.nn.sigmoid(
        jnp.dot(count_ref[...], b, preferred_element_type=jnp.float32))
    h_ref[...] = jnp.dot(x_ref[...], w_ref[...],
                         preferred_element_type=jnp.float32)


def _agg_sc_kernel(h_hbm, gflat, gate_idx, src2d, dst2d, part_out,
                   acc, rows, gbuf, gidxbuf, sbuf, dbuf, zbuf, gsem, ssem):
    c = lax.axis_index("c")
    s = lax.axis_index("s")
    wid = s * NC + c

    def zb(r, _):
        for f in range(D // 16):
            zbuf[r, pl.ds(f * 16, 16)] = jnp.zeros((16,), jnp.float32)
        return _
    lax.fori_loop(0, 48, zb, None)
    zbase = pl.multiple_of(s * ZROWS, 8)
    for k in range(ZROWS // 48):
        pltpu.sync_copy(zbuf, acc.at[pl.ds(zbase + k * 48, 48)])
    @pl.when(s == NS - 1)
    def _():
        pltpu.sync_copy(zbuf.at[pl.ds(0, 16)], acc.at[pl.ds(N - 16, 16)])
    plsc.subcore_barrier()

    def start_gather(j):
        b = j % 2
        return (
            pltpu.async_copy(h_hbm.at[sbuf.at[j]], rows.at[b], gsem),
            pltpu.async_copy(gflat.at[gidxbuf.at[j]],
                             gbuf.at[b, pl.ds(0, 100)], gsem),
        )

    def do_chunk(rowbase):
        pltpu.sync_copy(gate_idx.at[pl.ds(rowbase, C_CHUNK)], gidxbuf)
        pltpu.sync_copy(src2d.at[pl.ds(rowbase, C_CHUNK)], sbuf)
        pltpu.sync_copy(dst2d.at[pl.ds(rowbase, C_CHUNK)], dbuf)
        gds = [start_gather(0)]
        sds = []
        for j in range(C_CHUNK):
            b = j % 2
            if j < C_CHUNK - 1:
                if j >= 1:
                    sds[j - 1].wait()
                gds.append(start_gather(j + 1))
            gds[j][0].wait()
            gds[j][1].wait()

            def scale(r2, _):
                for rr in range(2):
                    r = r2 * 2 + rr
                    lane = r & 15
                    gv = gbuf[b, pl.ds(pl.multiple_of(r - lane, 16), 16)]
                    g = gv.at[jnp.full((16,), lane, jnp.int32)].get(
                        mode="promise_in_bounds")
                    for f in range(D // 16):
                        rows[b, r, pl.ds(f * 16, 16)] = (
                            rows[b, r, pl.ds(f * 16, 16)] * g)
                return _
            lax.fori_loop(0, 50, scale, None)

            sds.append(pltpu.async_copy(rows.at[b], acc.at[dbuf.at[j]],
                                        ssem, add=True))
        sds[-2].wait()
        sds[-1].wait()

    def chunk(k, _):
        do_chunk(pl.multiple_of((k * NW + wid) * C_CHUNK, 8))
        return _
    lax.fori_loop(0, C_STEPS, chunk, None)
    @pl.when(wid < C_EXTRA // C_CHUNK)
    def _():
        do_chunk(pl.multiple_of((C_STEPS * NW + wid) * C_CHUNK, 8))
    plsc.subcore_barrier()

    zbase2 = pl.multiple_of(s * ZROWS, 8)
    pltpu.sync_copy(acc.at[pl.ds(zbase2, ZROWS)],
                    part_out.at[c, pl.ds(zbase2, ZROWS)])
    @pl.when(s == NS - 1)
    def _():
        pltpu.sync_copy(acc.at[pl.ds(N - 16, 16)],
                        part_out.at[c, pl.ds(N - 16, 16)])


def _final_kernel(part_ref, out_ref):
    out_ref[...] = jnp.maximum(part_ref[0] + part_ref[1], 0.0)


@jax.jit
def kernel(x, edge_index, edge_type, W, G_sender, G_receiver):
    src2d = edge_index[0].reshape(ROWS, 100)
    dst2d = edge_index[1].reshape(ROWS, 100)
    et2d = edge_type.reshape(ROWS, 100)

    hist_idx, gate_idx = pl.pallas_call(
        _prep_kernel,
        out_shape=[
            jax.ShapeDtypeStruct((NC, ROWS, 100), jnp.int32),
            jax.ShapeDtypeStruct((ROWS, 100), jnp.int32),
        ],
    )(src2d, dst2d, et2d)

    mesh = plsc.VectorSubcoreMesh(core_axis_name="c", subcore_axis_name="s")

    count_raw = pl.kernel(
        _hist_sc_kernel,
        out_type=jax.ShapeDtypeStruct((NC * TBL,), jnp.float32),
        mesh=mesh,
        scratch_types=[
            pltpu.VMEM_SHARED((TBL,), jnp.float32),
            pltpu.VMEM((ROWS // NS, 100), jnp.int32),
            pltpu.VMEM((8192,), jnp.float32),
            pltpu.VMEM((112,), jnp.float32),
            pltpu.SemaphoreType.DMA,
        ],
    )(hist_idx)

    count = jnp.concatenate(
        [count_raw[:HALF_N * R], count_raw[TBL:TBL + HALF_N * R]]
    ).reshape(N, R)

    h, gate_table = pl.pallas_call(
        _dense_kernel,
        grid=(10,),
        in_specs=[
            pl.BlockSpec((N // 10, R), lambda m: (m, 0)),
            pl.BlockSpec((N // 10, D), lambda m: (m, 0)),
            pl.BlockSpec((D, D), lambda m: (0, 0)),
            pl.BlockSpec((R, 100), lambda m: (0, 0)),
            pl.BlockSpec((R, 100), lambda m: (0, 0)),
        ],
        out_specs=[
            pl.BlockSpec((N // 10, D), lambda m: (m, 0)),
            pl.BlockSpec((N // 10, R), lambda m: (m, 0)),
        ],
        out_shape=[
            jax.ShapeDtypeStruct((N, D), jnp.float32),
            jax.ShapeDtypeStruct((N, R), jnp.float32),
        ],
    )(count, x, W, G_sender, G_receiver)

    gflat = gate_table.reshape(N * R)

    partials = pl.kernel(
        _agg_sc_kernel,
        out_type=jax.ShapeDtypeStruct((NC, N, D), jnp.float32),
        mesh=mesh,
        scratch_types=[
            pltpu.VMEM_SHARED((N, D), jnp.float32),
            pltpu.VMEM((2, 100, D), jnp.float32),
            pltpu.VMEM((C_CHUNK, 128), jnp.float32),
            pltpu.VMEM((C_CHUNK, 100), jnp.int32),
            pltpu.VMEM((C_CHUNK, 100), jnp.int32),
            pltpu.VMEM((C_CHUNK, 100), jnp.int32),
            pltpu.VMEM((48, D), jnp.float32),
            pltpu.SemaphoreType.DMA,
            pltpu.SemaphoreType.DMA,
        ],
    )(h, gflat, gate_idx, src2d, dst2d)

    out = pl.pallas_call(
        _final_kernel,
        grid=(10,),
        in_specs=[pl.BlockSpec((NC, N // 10, D), lambda m: (0, m, 0))],
        out_specs=pl.BlockSpec((N // 10, D), lambda m: (m, 0)),
        out_shape=jax.ShapeDtypeStruct((N, D), jnp.float32),
    )(partials)
    return out


# speedup vs baseline: 12.7971x; 1.4850x over previous
import jax
import jax.numpy as jnp
from jax import lax
from jax.experimental import pallas as pl
from jax.experimental.pallas import tpu as pltpu
from jax.experimental.pallas import tpu_sc as plsc

N = 10000
E = 320000
D = 128
R = 200

NC = 2
NS = 16
NW = NC * NS

HALF_N = N // NC
TBL = 1 << 20
TRASH = HALF_N * R

ROWS = E // 100
A_CHUNK = 40
A_STEPS = ROWS // (NS * A_CHUNK)
C_CHUNK = 8
C_STEPS = ROWS // (NW * C_CHUNK)
C_EXTRA = ROWS - NW * C_CHUNK * C_STEPS

ZROWS = 624


def _prep_kernel(src_ref, dst_ref, et_ref, x_ref, w_ref,
                 hist_ref, gidx_ref, h_ref):
    src = src_ref[...]
    dst = dst_ref[...]
    t = et_ref[...]
    gidx_ref[...] = src * R + t
    spread = TRASH + (
        lax.broadcasted_iota(jnp.int32, (ROWS, 100), 0) * 100
        + lax.broadcasted_iota(jnp.int32, (ROWS, 100), 1)) % (TBL - TRASH)
    hist_ref[0] = jnp.where(dst < HALF_N, dst * R + t, spread)
    hist_ref[1] = jnp.where(dst >= HALF_N, (dst - HALF_N) * R + t, spread)
    h_ref[...] = jnp.dot(x_ref[...], w_ref[...],
                         preferred_element_type=jnp.float32)


def _hist_sc_kernel(hist_idx, count_out, table, idxbuf, zbuf, ones, sem):
    c = lax.axis_index("c")
    s = lax.axis_index("s")

    def zb(i, _):
        zbuf[pl.ds(pl.multiple_of(i * 16, 16), 16)] = jnp.zeros(
            (16,), jnp.float32)
        return _
    lax.fori_loop(0, 512, zb, None)
    for i in range(7):
        ones[pl.ds(i * 16, 16)] = jnp.ones((16,), jnp.float32)

    zwords = TBL // NS
    for k in range(zwords // 8192):
        pltpu.sync_copy(
            zbuf,
            table.at[pl.ds(pl.multiple_of(s * zwords + k * 8192, 8192), 8192)])
    plsc.subcore_barrier()

    rowbase = pl.multiple_of(s * (ROWS // NS), 8)
    pltpu.sync_copy(hist_idx.at[c, pl.ds(rowbase, ROWS // NS)], idxbuf)
    for w in range(ROWS // NS // A_CHUNK):
        ds = [pltpu.async_copy(ones.at[pl.ds(0, 100)],
                               table.at[idxbuf.at[w * A_CHUNK + j]],
                               sem, add=True)
              for j in range(A_CHUNK)]
        for d in ds:
            d.wait()
    plsc.subcore_barrier()

    base = pl.multiple_of(c * TBL + s * zwords, 8192)
    pltpu.sync_copy(table.at[pl.ds(pl.multiple_of(s * zwords, 8192), zwords)],
                    count_out.at[pl.ds(base, zwords)])


def _dense_kernel(count_ref, gs_ref, gr_ref, gate_ref):
    b = jnp.dot(gr_ref[...], gs_ref[...].T,
                preferred_element_type=jnp.float32)
    gate_ref[...] = jax.nn.sigmoid(
        jnp.dot(count_ref[...], b, preferred_element_type=jnp.float32))


def _agg_sc_kernel(h_hbm, gflat, gate_idx, src2d, dst2d, part_out,
                   acc, rows, gbuf, gidxbuf, sbuf, dbuf, zbuf, gsem, ssem):
    c = lax.axis_index("c")
    s = lax.axis_index("s")
    wid = s * NC + c

    def zb(r, _):
        for f in range(D // 16):
            zbuf[r, pl.ds(f * 16, 16)] = jnp.zeros((16,), jnp.float32)
        return _
    lax.fori_loop(0, 48, zb, None)
    zbase = pl.multiple_of(s * ZROWS, 8)
    for k in range(ZROWS // 48):
        pltpu.sync_copy(zbuf, acc.at[pl.ds(zbase + k * 48, 48)])
    @pl.when(s == NS - 1)
    def _():
        pltpu.sync_copy(zbuf.at[pl.ds(0, 16)], acc.at[pl.ds(N - 16, 16)])
    plsc.subcore_barrier()

    def start_gather(j):
        b = j % 2
        return (
            pltpu.async_copy(h_hbm.at[sbuf.at[j]], rows.at[b], gsem),
            pltpu.async_copy(gflat.at[gidxbuf.at[j]],
                             gbuf.at[b, pl.ds(0, 100)], gsem),
        )

    def do_chunk(rowbase):
        pltpu.sync_copy(gate_idx.at[pl.ds(rowbase, C_CHUNK)], gidxbuf)
        pltpu.sync_copy(src2d.at[pl.ds(rowbase, C_CHUNK)], sbuf)
        pltpu.sync_copy(dst2d.at[pl.ds(rowbase, C_CHUNK)], dbuf)
        gds = [start_gather(0)]
        sds = []
        for j in range(C_CHUNK):
            b = j % 2
            if j < C_CHUNK - 1:
                if j >= 1:
                    sds[j - 1].wait()
                gds.append(start_gather(j + 1))
            gds[j][0].wait()
            gds[j][1].wait()

            def scale(r2, _):
                for rr in range(2):
                    r = r2 * 2 + rr
                    lane = r & 15
                    gv = gbuf[b, pl.ds(pl.multiple_of(r - lane, 16), 16)]
                    g = gv.at[jnp.full((16,), lane, jnp.int32)].get(
                        mode="promise_in_bounds")
                    for f in range(D // 16):
                        rows[b, r, pl.ds(f * 16, 16)] = (
                            rows[b, r, pl.ds(f * 16, 16)] * g)
                return _
            lax.fori_loop(0, 50, scale, None)

            sds.append(pltpu.async_copy(rows.at[b], acc.at[dbuf.at[j]],
                                        ssem, add=True))
        sds[-2].wait()
        sds[-1].wait()

    def chunk(k, _):
        do_chunk(pl.multiple_of((k * NW + wid) * C_CHUNK, 8))
        return _
    lax.fori_loop(0, C_STEPS, chunk, None)
    @pl.when(wid < C_EXTRA // C_CHUNK)
    def _():
        do_chunk(pl.multiple_of((C_STEPS * NW + wid) * C_CHUNK, 8))
    plsc.subcore_barrier()

    zbase2 = pl.multiple_of(s * ZROWS, 8)
    pltpu.sync_copy(acc.at[pl.ds(zbase2, ZROWS)],
                    part_out.at[c, pl.ds(zbase2, ZROWS)])
    @pl.when(s == NS - 1)
    def _():
        pltpu.sync_copy(acc.at[pl.ds(N - 16, 16)],
                        part_out.at[c, pl.ds(N - 16, 16)])


def _final_kernel(part_ref, out_ref):
    out_ref[...] = jnp.maximum(part_ref[0] + part_ref[1], 0.0)


@jax.jit
def kernel(x, edge_index, edge_type, W, G_sender, G_receiver):
    src2d = edge_index[0].reshape(ROWS, 100)
    dst2d = edge_index[1].reshape(ROWS, 100)
    et2d = edge_type.reshape(ROWS, 100)

    hist_idx, gate_idx, h = pl.pallas_call(
        _prep_kernel,
        out_shape=[
            jax.ShapeDtypeStruct((NC, ROWS, 100), jnp.int32),
            jax.ShapeDtypeStruct((ROWS, 100), jnp.int32),
            jax.ShapeDtypeStruct((N, D), jnp.float32),
        ],
    )(src2d, dst2d, et2d, x, W)

    mesh = plsc.VectorSubcoreMesh(core_axis_name="c", subcore_axis_name="s")

    count_raw = pl.kernel(
        _hist_sc_kernel,
        out_type=jax.ShapeDtypeStruct((NC * TBL,), jnp.float32),
        mesh=mesh,
        scratch_types=[
            pltpu.VMEM_SHARED((TBL,), jnp.float32),
            pltpu.VMEM((ROWS // NS, 100), jnp.int32),
            pltpu.VMEM((8192,), jnp.float32),
            pltpu.VMEM((112,), jnp.float32),
            pltpu.SemaphoreType.DMA,
        ],
    )(hist_idx)

    count = jnp.concatenate(
        [count_raw[:HALF_N * R], count_raw[TBL:TBL + HALF_N * R]]
    ).reshape(N, R)

    gate_table = pl.pallas_call(
        _dense_kernel,
        grid=(10,),
        in_specs=[
            pl.BlockSpec((N // 10, R), lambda m: (m, 0)),
            pl.BlockSpec((R, 100), lambda m: (0, 0)),
            pl.BlockSpec((R, 100), lambda m: (0, 0)),
        ],
        out_specs=pl.BlockSpec((N // 10, R), lambda m: (m, 0)),
        out_shape=jax.ShapeDtypeStruct((N, R), jnp.float32),
    )(count, G_sender, G_receiver)

    gflat = gate_table.reshape(N * R)

    partials = pl.kernel(
        _agg_sc_kernel,
        out_type=jax.ShapeDtypeStruct((NC, N, D), jnp.float32),
        mesh=mesh,
        scratch_types=[
            pltpu.VMEM_SHARED((N, D), jnp.float32),
            pltpu.VMEM((2, 100, D), jnp.float32),
            pltpu.VMEM((C_CHUNK, 128), jnp.float32),
            pltpu.VMEM((C_CHUNK, 100), jnp.int32),
            pltpu.VMEM((C_CHUNK, 100), jnp.int32),
            pltpu.VMEM((C_CHUNK, 100), jnp.int32),
            pltpu.VMEM((48, D), jnp.float32),
            pltpu.SemaphoreType.DMA,
            pltpu.SemaphoreType.DMA,
        ],
    )(h, gflat, gate_idx, src2d, dst2d)

    out = pl.pallas_call(
        _final_kernel,
        grid=(10,),
        in_specs=[pl.BlockSpec((NC, N // 10, D), lambda m: (0, m, 0))],
        out_specs=pl.BlockSpec((N // 10, D), lambda m: (m, 0)),
        out_shape=jax.ShapeDtypeStruct((N, D), jnp.float32),
    )(partials)
    return out


# speedup vs baseline: 12.9666x; 1.0132x over previous
import jax
import jax.numpy as jnp
from jax import lax
from jax.experimental import pallas as pl
from jax.experimental.pallas import tpu as pltpu
from jax.experimental.pallas import tpu_sc as plsc

N = 10000
E = 320000
D = 128
R = 200

NC = 2
NS = 16
NW = NC * NS

HALF_N = N // NC
TBL = 1 << 20
TRASH = HALF_N * R

ROWS = E // 100
A_CHUNK = 40
A_STEPS = ROWS // (NS * A_CHUNK)
C_CHUNK = 8
C_STEPS = ROWS // (NW * C_CHUNK)
C_EXTRA = ROWS - NW * C_CHUNK * C_STEPS

ZROWS = 624


def _prep_kernel(src_ref, dst_ref, et_ref, x_ref, w_ref,
                 hist_ref, gidx_ref, h_ref):
    src = src_ref[...]
    dst = dst_ref[...]
    t = et_ref[...]
    gidx_ref[...] = src * R + t
    blk = src.shape[0]
    spread = TRASH + (
        pl.program_id(0) * (blk * 100)
        + lax.broadcasted_iota(jnp.int32, (blk, 100), 0) * 100
        + lax.broadcasted_iota(jnp.int32, (blk, 100), 1)) % (TBL - TRASH)
    hist_ref[0] = jnp.where(dst < HALF_N, dst * R + t, spread)
    hist_ref[1] = jnp.where(dst >= HALF_N, (dst - HALF_N) * R + t, spread)
    h_ref[...] = jnp.dot(x_ref[...], w_ref[...],
                         preferred_element_type=jnp.float32)


def _hist_sc_kernel(hist_idx, count_out, table, idxbuf, zbuf, ones, sem):
    c = lax.axis_index("c")
    s = lax.axis_index("s")

    def zb(i, _):
        zbuf[pl.ds(pl.multiple_of(i * 16, 16), 16)] = jnp.zeros(
            (16,), jnp.float32)
        return _
    lax.fori_loop(0, 512, zb, None)
    for i in range(7):
        ones[pl.ds(i * 16, 16)] = jnp.ones((16,), jnp.float32)

    zwords = TBL // NS
    for k in range(zwords // 8192):
        pltpu.sync_copy(
            zbuf,
            table.at[pl.ds(pl.multiple_of(s * zwords + k * 8192, 8192), 8192)])
    plsc.subcore_barrier()

    rowbase = pl.multiple_of(s * (ROWS // NS), 8)
    pltpu.sync_copy(hist_idx.at[c, pl.ds(rowbase, ROWS // NS)], idxbuf)
    for w in range(ROWS // NS // A_CHUNK):
        ds = [pltpu.async_copy(ones.at[pl.ds(0, 100)],
                               table.at[idxbuf.at[w * A_CHUNK + j]],
                               sem, add=True)
              for j in range(A_CHUNK)]
        for d in ds:
            d.wait()
    plsc.subcore_barrier()

    base = pl.multiple_of(c * TBL + s * zwords, 8192)
    pltpu.sync_copy(table.at[pl.ds(pl.multiple_of(s * zwords, 8192), zwords)],
                    count_out.at[pl.ds(base, zwords)])


def _dense_kernel(count_ref, gs_ref, gr_ref, gate_ref):
    b = jnp.dot(gr_ref[...], gs_ref[...].T,
                preferred_element_type=jnp.float32)
    gate_ref[...] = jax.nn.sigmoid(
        jnp.dot(count_ref[...], b, preferred_element_type=jnp.float32))


def _agg_sc_kernel(h_hbm, gflat, gate_idx, src2d, dst2d, part_out,
                   acc, rows, gbuf, gidxbuf, sbuf, dbuf, zbuf, gsem, ssem):
    c = lax.axis_index("c")
    s = lax.axis_index("s")
    wid = s * NC + c

    def zb(r, _):
        for f in range(D // 16):
            zbuf[r, pl.ds(f * 16, 16)] = jnp.zeros((16,), jnp.float32)
        return _
    lax.fori_loop(0, 24, zb, None)
    zbase = pl.multiple_of(s * ZROWS, 8)
    for k in range(ZROWS // 24):
        pltpu.sync_copy(zbuf, acc.at[pl.ds(zbase + k * 24, 24)])
    @pl.when(s == NS - 1)
    def _():
        pltpu.sync_copy(zbuf.at[pl.ds(0, 16)], acc.at[pl.ds(N - 16, 16)])
    plsc.subcore_barrier()

    NB = 3

    def start_gather(j):
        b = j % NB
        return (
            pltpu.async_copy(h_hbm.at[sbuf.at[j]], rows.at[b], gsem),
            pltpu.async_copy(gflat.at[gidxbuf.at[j]],
                             gbuf.at[b, pl.ds(0, 100)], gsem),
        )

    def do_chunk(rowbase):
        pltpu.sync_copy(gate_idx.at[pl.ds(rowbase, C_CHUNK)], gidxbuf)
        pltpu.sync_copy(src2d.at[pl.ds(rowbase, C_CHUNK)], sbuf)
        pltpu.sync_copy(dst2d.at[pl.ds(rowbase, C_CHUNK)], dbuf)
        gds = [start_gather(0), start_gather(1)]
        sds = []
        for j in range(C_CHUNK):
            b = j % NB
            if j + 2 < C_CHUNK:
                if j >= 1:
                    sds[j - 1].wait()
                gds.append(start_gather(j + 2))
            gds[j][0].wait()
            gds[j][1].wait()

            def scale(r2, _):
                for rr in range(2):
                    r = r2 * 2 + rr
                    lane = r & 15
                    gv = gbuf[b, pl.ds(pl.multiple_of(r - lane, 16), 16)]
                    g = gv.at[jnp.full((16,), lane, jnp.int32)].get(
                        mode="promise_in_bounds")
                    for f in range(D // 16):
                        rows[b, r, pl.ds(f * 16, 16)] = (
                            rows[b, r, pl.ds(f * 16, 16)] * g)
                return _
            lax.fori_loop(0, 50, scale, None)

            sds.append(pltpu.async_copy(rows.at[b], acc.at[dbuf.at[j]],
                                        ssem, add=True))
        for d in sds[-(NB - 1) - 1:]:
            d.wait()

    def chunk(k, _):
        do_chunk(pl.multiple_of((k * NW + wid) * C_CHUNK, 8))
        return _
    lax.fori_loop(0, C_STEPS, chunk, None)
    @pl.when(wid < C_EXTRA // C_CHUNK)
    def _():
        do_chunk(pl.multiple_of((C_STEPS * NW + wid) * C_CHUNK, 8))
    plsc.subcore_barrier()

    zbase2 = pl.multiple_of(s * ZROWS, 8)
    pltpu.sync_copy(acc.at[pl.ds(zbase2, ZROWS)],
                    part_out.at[c, pl.ds(zbase2, ZROWS)])
    @pl.when(s == NS - 1)
    def _():
        pltpu.sync_copy(acc.at[pl.ds(N - 16, 16)],
                        part_out.at[c, pl.ds(N - 16, 16)])


def _final_kernel(part_ref, out_ref):
    out_ref[...] = jnp.maximum(part_ref[0] + part_ref[1], 0.0)


@jax.jit
def kernel(x, edge_index, edge_type, W, G_sender, G_receiver):
    src2d = edge_index[0].reshape(ROWS, 100)
    dst2d = edge_index[1].reshape(ROWS, 100)
    et2d = edge_type.reshape(ROWS, 100)

    hist_idx, gate_idx, h = pl.pallas_call(
        _prep_kernel,
        grid=(10,),
        in_specs=[
            pl.BlockSpec((ROWS // 10, 100), lambda m: (m, 0)),
            pl.BlockSpec((ROWS // 10, 100), lambda m: (m, 0)),
            pl.BlockSpec((ROWS // 10, 100), lambda m: (m, 0)),
            pl.BlockSpec((N // 10, D), lambda m: (m, 0)),
            pl.BlockSpec((D, D), lambda m: (0, 0)),
        ],
        out_specs=[
            pl.BlockSpec((NC, ROWS // 10, 100), lambda m: (0, m, 0)),
            pl.BlockSpec((ROWS // 10, 100), lambda m: (m, 0)),
            pl.BlockSpec((N // 10, D), lambda m: (m, 0)),
        ],
        out_shape=[
            jax.ShapeDtypeStruct((NC, ROWS, 100), jnp.int32),
            jax.ShapeDtypeStruct((ROWS, 100), jnp.int32),
            jax.ShapeDtypeStruct((N, D), jnp.float32),
        ],
    )(src2d, dst2d, et2d, x, W)

    mesh = plsc.VectorSubcoreMesh(core_axis_name="c", subcore_axis_name="s")

    count_raw = pl.kernel(
        _hist_sc_kernel,
        out_type=jax.ShapeDtypeStruct((NC * TBL,), jnp.float32),
        mesh=mesh,
        scratch_types=[
            pltpu.VMEM_SHARED((TBL,), jnp.float32),
            pltpu.VMEM((ROWS // NS, 100), jnp.int32),
            pltpu.VMEM((8192,), jnp.float32),
            pltpu.VMEM((112,), jnp.float32),
            pltpu.SemaphoreType.DMA,
        ],
    )(hist_idx)

    count = jnp.concatenate(
        [count_raw[:HALF_N * R], count_raw[TBL:TBL + HALF_N * R]]
    ).reshape(N, R)

    gate_table = pl.pallas_call(
        _dense_kernel,
        grid=(10,),
        in_specs=[
            pl.BlockSpec((N // 10, R), lambda m: (m, 0)),
            pl.BlockSpec((R, 100), lambda m: (0, 0)),
            pl.BlockSpec((R, 100), lambda m: (0, 0)),
        ],
        out_specs=pl.BlockSpec((N // 10, R), lambda m: (m, 0)),
        out_shape=jax.ShapeDtypeStruct((N, R), jnp.float32),
    )(count, G_sender, G_receiver)

    gflat = gate_table.reshape(N * R)

    partials = pl.kernel(
        _agg_sc_kernel,
        out_type=jax.ShapeDtypeStruct((NC, N, D), jnp.float32),
        mesh=mesh,
        scratch_types=[
            pltpu.VMEM_SHARED((N, D), jnp.float32),
            pltpu.VMEM((3, 100, D), jnp.float32),
            pltpu.VMEM((3, 128), jnp.float32),
            pltpu.VMEM((C_CHUNK, 100), jnp.int32),
            pltpu.VMEM((C_CHUNK, 100), jnp.int32),
            pltpu.VMEM((C_CHUNK, 100), jnp.int32),
            pltpu.VMEM((24, D), jnp.float32),
            pltpu.SemaphoreType.DMA,
            pltpu.SemaphoreType.DMA,
        ],
    )(h, gflat, gate_idx, src2d, dst2d)

    out = pl.pallas_call(
        _final_kernel,
        grid=(10,),
        in_specs=[pl.BlockSpec((NC, N // 10, D), lambda m: (0, m, 0))],
        out_specs=pl.BlockSpec((N // 10, D), lambda m: (m, 0)),
        out_shape=jax.ShapeDtypeStruct((N, D), jnp.float32),
    )(partials)
    return out
